# baseline (speedup 1.0000x reference)
import functools
from itertools import combinations

import jax
import jax.numpy as jnp
import numpy as np
from jax import lax
from jax.experimental import pallas as pl
from jax.experimental.pallas import tpu as pltpu
from jax.experimental.pallas import tpu_sc as plsc

B, F, V, D = 4096, 26, 1000000, 16
FD = F * D
BF = B * F
MLP_W = 256
NUM_PAIRS = F * (F - 1) // 2

NC, NS = 2, 16
NW = NC * NS

NT_TILES = (V + 127) // 128
TPW = 244
NTS = 28
NBINS = 9
CAPB = 640
OUTW = NBINS * CAPB
CAP_TOT = NW * OUTW
M_CAP = 4608
SCCHUNK = 2048
NSCCH = BF // SCCHUNK
NCH_B = OUTW // 128
DUMPN = 104
DUMP0 = BF
CHUNK_W = BF // NW

_PAIR_ROWS, _PAIR_COLS = map(np.array, zip(*combinations(range(F), 2)))
_IOTA16 = np.arange(16, dtype=np.int32)


def _splat(x):
    return jnp.full((16,), x, dtype=jnp.int32)


def _sc_stream_body(idx_hbm, vt_hbm, mval_out, pos_out,
                    idx_st, r_list, p_list, rbin, pbin, slab,
                    vstage, sem_t):
    wid = lax.axis_index("s") * NC + lax.axis_index("c")
    ja = wid * TPW
    jb = jnp.where(wid == NW - 1, NT_TILES, ja + TPW)
    iota = lax.iota(jnp.int32, 16)

    def scan_chunk(c, cnt):
        pltpu.sync_copy(idx_hbm.at[pl.ds(c * SCCHUNK, SCCHUNK)], idx_st)

        def scan_vec(i, cnt):
            rvec = idx_st[pl.ds(i * 16, 16)]
            j7 = lax.shift_right_logical(rvec, 7)
            m = (j7 >= _splat(ja)) & (j7 < _splat(jb))
            posvec = _splat(c * SCCHUNK + i * 16) + iota
            inc = plsc.cumsum(m.astype(jnp.int32))
            dest = jnp.minimum(cnt + inc - 1, _splat(M_CAP - 1))
            plsc.store_scatter(r_list, [dest], rvec, mask=m)
            plsc.store_scatter(p_list, [dest], posvec, mask=m)
            return cnt + plsc.all_reduce_population_count(m)

        return lax.fori_loop(0, SCCHUNK // 16, scan_vec, cnt)

    cnt_vec = lax.fori_loop(0, NSCCH, scan_chunk, jnp.zeros((16,), jnp.int32))
    cnt = jnp.max(cnt_vec, axis=0)

    def fill_dump(i, carry):
        dv = _splat(DUMP0) + iota + _splat(16 * (i % 6))
        pbin[pl.ds(i * 16, 16)] = dv
        return carry

    lax.fori_loop(0, NBINS * CAPB // 16, fill_dump, 0)

    def bin_vec(i, cnts):
        rvec = r_list[pl.ds(i * 16, 16)]
        pvec = p_list[pl.ds(i * 16, 16)]
        valid = (_splat(i * 16) + iota) < cnt_vec
        bb = (lax.shift_right_logical(rvec, 7) - _splat(ja)) // NTS
        new = []
        for k in range(NBINS):
            ck = cnts[k]
            mk = valid & (bb == _splat(k))
            inc = plsc.cumsum(mk.astype(jnp.int32))
            dest = jnp.minimum(ck + inc - 1, _splat(CAPB - 1)) + _splat(k * CAPB)
            plsc.store_scatter(rbin, [dest], rvec, mask=mk)
            plsc.store_scatter(pbin, [dest], pvec, mask=mk)
            new.append(ck + plsc.all_reduce_population_count(mk))
        return tuple(new)

    cnts_vec = lax.fori_loop(0, (cnt + 15) // 16, bin_vec,
                             tuple(jnp.zeros((16,), jnp.int32)
                                   for _ in range(NBINS)))
    cnts = [jnp.max(cv, axis=0) for cv in cnts_vec]

    for k in range(NBINS):
        j0 = ja + k * NTS
        ntk = jnp.maximum(jnp.minimum(jb - j0, NTS), 0)

        def fire(i, carry, j0=j0):
            jj = j0 + lax.shift_right_logical(i, 1)
            t = i & 1
            row = (lax.shift_right_logical(i, 1) * 16 + t * 8)
            pltpu.async_copy(
                vt_hbm.at[pl.ds(t * 8, 8), pl.ds(pl.multiple_of(jj * 128, 128), 128)],
                slab.at[pl.ds(pl.multiple_of(row, 8), 8), :], sem_t)
            return carry

        def drain(i, carry):
            pltpu.make_async_copy(
                vt_hbm.at[pl.ds(0, 8), pl.ds(0, 128)],
                slab.at[pl.ds(0, 8), :], sem_t).wait()
            return carry

        lax.fori_loop(0, 2 * ntk, fire, 0)
        lax.fori_loop(0, 2 * ntk, drain, 0)

        def extract(m, carry, k=k, j0=j0):
            rsp = plsc.load_gather(rbin, [_splat(k * CAPB + m)])
            jloc = lax.shift_right_logical(rsp, 7) - _splat(j0)
            rows = jloc * 16 + iota
            cols = rsp & _splat(127)
            v16 = plsc.load_gather(slab, [rows, cols])
            vstage[pl.ds(m * 16, 16)] = v16
            return carry

        lax.fori_loop(0, cnts[k], extract, 0)
        base = (wid * NBINS + k) * CAPB
        pltpu.sync_copy(vstage, mval_out.at[pl.ds(base * 16, CAPB * 16)])
        pltpu.sync_copy(pbin.at[pl.ds(k * CAPB, CAPB)],
                        pos_out.at[pl.ds(base, CAPB)])


@functools.cache
def _sc_stream():
    return pl.kernel(
        _sc_stream_body,
        out_type=[
            jax.ShapeDtypeStruct((CAP_TOT * 16,), jnp.float32),
            jax.ShapeDtypeStruct((CAP_TOT,), jnp.int32),
        ],
        mesh=plsc.VectorSubcoreMesh(core_axis_name="c", subcore_axis_name="s"),
        compiler_params=pltpu.CompilerParams(needs_layout_passes=False),
        scratch_types=[
            pltpu.VMEM((SCCHUNK,), jnp.int32),
            pltpu.VMEM((M_CAP,), jnp.int32),
            pltpu.VMEM((M_CAP,), jnp.int32),
            pltpu.VMEM((NBINS * CAPB,), jnp.int32),
            pltpu.VMEM((NBINS * CAPB,), jnp.int32),
            pltpu.VMEM((NTS * 16, 128), jnp.float32),
            pltpu.VMEM((CAPB * 16,), jnp.float32),
            pltpu.SemaphoreType.DMA,
        ],
    )


BLK = 15
NBLK = NCH_B // BLK


def _sc_scatter_body(mval_hbm, pos_hbm, idx_hbm, w16_hbm, val_hbm,
                     xv_out, lin_out,
                     pos_v, stv, idx_v, idxhi_v, w64_v, val_v, prod_v, lin_v,
                     sem_v, sem_g):
    wid = lax.axis_index("s") * NC + lax.axis_index("c")
    iota = lax.iota(jnp.int32, 16)
    fiota = iota.astype(jnp.float32)

    pltpu.sync_copy(pos_hbm.at[pl.ds(wid * NCH_B, NCH_B)], pos_v)

    def block(b, carry):
        base = wid * OUTW + b * BLK * 128
        pltpu.sync_copy(mval_hbm.at[pl.ds(base, BLK * 128)], stv)

        def fire(i, c):
            off = pl.multiple_of(i * 128, 128)
            pltpu.async_copy(stv.at[pl.ds(off, 128)],
                             xv_out.at[pos_v.at[b * BLK + i]], sem_v)
            return c

        def drain(i, c):
            pltpu.make_async_copy(stv.at[pl.ds(0, 128)],
                                  xv_out.at[pl.ds(0, 128)], sem_v).wait()
            return c

        lax.fori_loop(0, BLK, fire, 0)
        lax.fori_loop(0, BLK, drain, 0)
        return carry

    lax.fori_loop(0, NBLK, block, 0)

    cbase = wid * CHUNK_W
    pltpu.sync_copy(idx_hbm.at[pl.ds(cbase, CHUNK_W)], idx_v)
    pltpu.sync_copy(val_hbm.at[pl.ds(cbase, CHUNK_W)], val_v)

    def prep(j, carry):
        off = pl.multiple_of(j * 16, 16)
        idxhi_v[pl.ds(off, 16)] = lax.shift_right_logical(idx_v[pl.ds(off, 16)], 4)
        return carry

    lax.fori_loop(0, CHUNK_W // 16, prep, 0)

    def gchunk(j, carry):
        off = pl.multiple_of(j * 128, 128)
        pltpu.async_copy(w16_hbm.at[idxhi_v.at[pl.ds(off, 128)]],
                         w64_v.at[pl.ds(off, 128)], sem_g).wait()
        return carry

    lax.fori_loop(0, CHUNK_W // 128, gchunk, 0)

    def wmul(j, carry):
        off = pl.multiple_of(j * 16, 16)
        lanes = idx_v[pl.ds(off, 16)] & _splat(15)
        wv = plsc.load_gather(w64_v, [_splat(j * 16) + iota, lanes])
        prod_v[pl.ds(off, 16)] = wv * val_v[pl.ds(off, 16)]
        return carry

    lax.fori_loop(0, CHUNK_W // 16, wmul, 0)

    tailmask = (fiota < 10.0).astype(jnp.float32)
    lane0 = iota == 0

    def rowsum2(r, carry):
        s1 = prod_v[pl.ds(r * F, 16)]
        s2 = prod_v[pl.ds(r * F + 16, 16)] * tailmask
        tot = jnp.sum(s1 + s2, axis=0)
        plsc.store_scatter(lin_v, [_splat(r)],
                           jnp.full((16,), tot, jnp.float32), mask=lane0)
        return carry

    lax.fori_loop(0, CHUNK_W // F, rowsum2, 0)
    pltpu.sync_copy(lin_v, lin_out.at[pl.ds(wid * (CHUNK_W // F), CHUNK_W // F)])


@functools.cache
def _sc_scatter():
    return pl.kernel(
        _sc_scatter_body,
        out_type=[
            jax.ShapeDtypeStruct((BF + DUMPN, 16), jnp.float32),
            jax.ShapeDtypeStruct((B,), jnp.float32),
        ],
        mesh=plsc.VectorSubcoreMesh(core_axis_name="c", subcore_axis_name="s"),
        compiler_params=pltpu.CompilerParams(use_tc_tiling_on_sc=False,
                                             needs_layout_passes=False),
        scratch_types=[
            pltpu.VMEM((NCH_B, 128), jnp.int32),
            pltpu.VMEM((BLK * 128, 16), jnp.float32),
            pltpu.VMEM((CHUNK_W,), jnp.int32),
            pltpu.VMEM((CHUNK_W,), jnp.int32),
            pltpu.VMEM((CHUNK_W, 16), jnp.float32),
            pltpu.VMEM((CHUNK_W,), jnp.float32),
            pltpu.VMEM((CHUNK_W + 16,), jnp.float32),
            pltpu.VMEM((CHUNK_W // F,), jnp.float32),
            pltpu.SemaphoreType.DMA,
            pltpu.SemaphoreType.DMA,
        ],
    )


BB = 512
GRID = B // BB


def _tc_body(xv_ref, fv_ref, lin_ref, w0_ref, b0_ref, w1_ref, b1_ref,
             w2_ref, b2_ref, w3t_ref, k_ref, e_ref, sc_ref, out_ref):
    fv = fv_ref[...]
    val_exp = jnp.dot(fv, e_ref[...], preferred_element_type=jnp.float32)
    xv = xv_ref[...] * val_exp
    h = jnp.maximum(jnp.dot(xv, w0_ref[...], preferred_element_type=jnp.float32)
                    + b0_ref[...], 0.0)
    h = jnp.maximum(jnp.dot(h, w1_ref[...], preferred_element_type=jnp.float32)
                    + b1_ref[...], 0.0)
    h = jnp.maximum(jnp.dot(h, w2_ref[...], preferred_element_type=jnp.float32)
                    + b2_ref[...], 0.0)
    deep = jnp.sum(h * w3t_ref[...], axis=1, keepdims=True)
    z = jnp.dot(xv, k_ref[...], preferred_element_type=jnp.float32)
    fm = 0.5 * jnp.sum(xv * z, axis=1, keepdims=True)
    out_ref[...] = (lin_ref[...] + fm + deep) * sc_ref[0] + sc_ref[1]


_tc_call = pl.pallas_call(
    _tc_body,
    grid=(GRID,),
    in_specs=[
        pl.BlockSpec((BB, FD), lambda i: (i, 0)),
        pl.BlockSpec((BB, F), lambda i: (i, 0)),
        pl.BlockSpec((BB, 1), lambda i: (i, 0)),
        pl.BlockSpec((FD, MLP_W), lambda i: (0, 0)),
        pl.BlockSpec((1, MLP_W), lambda i: (0, 0)),
        pl.BlockSpec((MLP_W, MLP_W), lambda i: (0, 0)),
        pl.BlockSpec((1, MLP_W), lambda i: (0, 0)),
        pl.BlockSpec((MLP_W, MLP_W), lambda i: (0, 0)),
        pl.BlockSpec((1, MLP_W), lambda i: (0, 0)),
        pl.BlockSpec((1, MLP_W), lambda i: (0, 0)),
        pl.BlockSpec((FD, FD), lambda i: (0, 0)),
        pl.BlockSpec((F, FD), lambda i: (0, 0)),
        pl.BlockSpec(memory_space=pltpu.SMEM),
    ],
    out_specs=pl.BlockSpec((BB, 1), lambda i: (i, 0)),
    out_shape=jax.ShapeDtypeStruct((B, 1), jnp.float32),
)

_E = np.kron(np.eye(F, dtype=np.float32), np.ones((1, D), dtype=np.float32))


def kernel(feat_index, feat_value, w_table, v_table, W0, b0, W1, b1, W2, b2,
           W3, b3, mask, bn_gamma, bn_beta, W_out, b_out):
    idx_flat = feat_index.reshape(BF)
    vt = v_table.T
    w16 = w_table.reshape(V // 16, 16)
    val_flat = feat_value.reshape(BF)
    mval, pos = _sc_stream()(idx_flat, vt)
    xv_all, lin_sc = _sc_scatter()(
        mval.reshape(CAP_TOT, 16), pos.reshape(NW * NCH_B, 128),
        idx_flat, w16, val_flat)
    xv2 = xv_all.reshape((BF + DUMPN) // F, FD)[:B]
    lin2 = lin_sc.reshape(B, 1)

    coef = (mask[0] * bn_gamma) * (1.0 / np.sqrt(1.0 + 1e-3))
    M = jnp.zeros((F, F), jnp.float32).at[_PAIR_ROWS, _PAIR_COLS].set(coef)
    M = M + M.T
    K = jnp.kron(M, jnp.eye(D, dtype=jnp.float32))
    scale = W_out[0, 0]
    shift = b_out[0] + scale * (b3[0] + jnp.sum(mask[0] * bn_beta))
    sc = jnp.stack([scale, shift])

    return _tc_call(
        xv2, feat_value, lin2, W0, b0.reshape(1, MLP_W), W1,
        b1.reshape(1, MLP_W), W2, b2.reshape(1, MLP_W), W3.reshape(1, MLP_W),
        K, _E, sc)


# speedup vs baseline: 1.1910x; 1.1910x over previous
import functools
from itertools import combinations

import jax
import jax.numpy as jnp
import numpy as np
from jax import lax
from jax.experimental import pallas as pl
from jax.experimental.pallas import tpu as pltpu
from jax.experimental.pallas import tpu_sc as plsc

B, F, V, D = 4096, 26, 1000000, 16
FD = F * D
BF = B * F
MLP_W = 256
NUM_PAIRS = F * (F - 1) // 2

NC, NS = 2, 16
NW = NC * NS

NT_TILES = (V + 127) // 128
TPW = 244
NTS = 28
NBINS = 9
CAPB = 640
OUTW = NBINS * CAPB
CAP_TOT = NW * OUTW
M_CAP = 4608
SCCHUNK = 2048
NSCCH = BF // SCCHUNK
NCH_B = OUTW // 128
DUMPN = 104
DUMP0 = BF
CHUNK_W = BF // NW

_PAIR_ROWS, _PAIR_COLS = map(np.array, zip(*combinations(range(F), 2)))
_IOTA16 = np.arange(16, dtype=np.int32)


def _splat(x):
    return jnp.full((16,), x, dtype=jnp.int32)


def _sc_stream_body(idx_hbm, vt_hbm, mval_out, pos_out,
                    idx_st, r_list, p_list, rbin, pbin, slab,
                    vstage, sem_t):
    wid = lax.axis_index("s") * NC + lax.axis_index("c")
    ja = wid * TPW
    jb = jnp.where(wid == NW - 1, NT_TILES, ja + TPW)
    iota = lax.iota(jnp.int32, 16)

    def scan_chunk(c, cnt):
        pltpu.sync_copy(idx_hbm.at[pl.ds(c * SCCHUNK, SCCHUNK)], idx_st)

        def scan_vec(i, cnt):
            rvec = idx_st[pl.ds(i * 16, 16)]
            j7 = lax.shift_right_logical(rvec, 7)
            m = (j7 >= _splat(ja)) & (j7 < _splat(jb))
            posvec = _splat(c * SCCHUNK + i * 16) + iota
            inc = plsc.cumsum(m.astype(jnp.int32))
            dest = jnp.minimum(cnt + inc - 1, _splat(M_CAP - 1))
            plsc.store_scatter(r_list, [dest], rvec, mask=m)
            plsc.store_scatter(p_list, [dest], posvec, mask=m)
            return cnt + plsc.all_reduce_population_count(m)

        return lax.fori_loop(0, SCCHUNK // 16, scan_vec, cnt)

    cnt_vec = lax.fori_loop(0, NSCCH, scan_chunk, jnp.zeros((16,), jnp.int32))
    cnt = jnp.max(cnt_vec, axis=0)

    def fill_dump(i, carry):
        dv = _splat(DUMP0) + iota + _splat(16 * (i % 6))
        pbin[pl.ds(i * 16, 16)] = dv
        return carry

    lax.fori_loop(0, NBINS * CAPB // 16, fill_dump, 0)

    def bin_vec(i, cnts):
        rvec = r_list[pl.ds(i * 16, 16)]
        pvec = p_list[pl.ds(i * 16, 16)]
        valid = (_splat(i * 16) + iota) < cnt_vec
        bb = (lax.shift_right_logical(rvec, 7) - _splat(ja)) // NTS
        new = []
        for k in range(NBINS):
            ck = cnts[k]
            mk = valid & (bb == _splat(k))
            inc = plsc.cumsum(mk.astype(jnp.int32))
            dest = jnp.minimum(ck + inc - 1, _splat(CAPB - 1)) + _splat(k * CAPB)
            plsc.store_scatter(rbin, [dest], rvec, mask=mk)
            plsc.store_scatter(pbin, [dest], pvec, mask=mk)
            new.append(ck + plsc.all_reduce_population_count(mk))
        return tuple(new)

    cnts_vec = lax.fori_loop(0, (cnt + 15) // 16, bin_vec,
                             tuple(jnp.zeros((16,), jnp.int32)
                                   for _ in range(NBINS)))
    cnts = [jnp.max(cv, axis=0) for cv in cnts_vec]

    for k in range(NBINS):
        j0 = ja + k * NTS
        ntk = jnp.maximum(jnp.minimum(jb - j0, NTS), 0)

        def fire(i, carry, j0=j0):
            jj = j0 + lax.shift_right_logical(i, 1)
            t = i & 1
            row = (lax.shift_right_logical(i, 1) * 16 + t * 8)
            pltpu.async_copy(
                vt_hbm.at[pl.ds(t * 8, 8), pl.ds(pl.multiple_of(jj * 128, 128), 128)],
                slab.at[pl.ds(pl.multiple_of(row, 8), 8), :], sem_t)
            return carry

        def drain(i, carry):
            pltpu.make_async_copy(
                vt_hbm.at[pl.ds(0, 8), pl.ds(0, 128)],
                slab.at[pl.ds(0, 8), :], sem_t).wait()
            return carry

        lax.fori_loop(0, 2 * ntk, fire, 0)
        lax.fori_loop(0, 2 * ntk, drain, 0)

        def extract(m, carry, k=k, j0=j0):
            rsp = plsc.load_gather(rbin, [_splat(k * CAPB + m)])
            jloc = lax.shift_right_logical(rsp, 7) - _splat(j0)
            rows = jloc * 16 + iota
            cols = rsp & _splat(127)
            v16 = plsc.load_gather(slab, [rows, cols])
            vstage[pl.ds(m * 16, 16)] = v16
            return carry

        lax.fori_loop(0, cnts[k], extract, 0)
        base = (wid * NBINS + k) * CAPB
        pltpu.sync_copy(vstage, mval_out.at[pl.ds(base * 16, CAPB * 16)])
        pltpu.sync_copy(pbin.at[pl.ds(k * CAPB, CAPB)],
                        pos_out.at[pl.ds(base, CAPB)])


@functools.cache
def _sc_stream():
    return pl.kernel(
        _sc_stream_body,
        out_type=[
            jax.ShapeDtypeStruct((CAP_TOT * 16,), jnp.float32),
            jax.ShapeDtypeStruct((CAP_TOT,), jnp.int32),
        ],
        mesh=plsc.VectorSubcoreMesh(core_axis_name="c", subcore_axis_name="s"),
        compiler_params=pltpu.CompilerParams(needs_layout_passes=False),
        scratch_types=[
            pltpu.VMEM((SCCHUNK,), jnp.int32),
            pltpu.VMEM((M_CAP,), jnp.int32),
            pltpu.VMEM((M_CAP,), jnp.int32),
            pltpu.VMEM((NBINS * CAPB,), jnp.int32),
            pltpu.VMEM((NBINS * CAPB,), jnp.int32),
            pltpu.VMEM((NTS * 16, 128), jnp.float32),
            pltpu.VMEM((CAPB * 16,), jnp.float32),
            pltpu.SemaphoreType.DMA,
        ],
    )


BLK = 15
NBLK = NCH_B // BLK


def _sc_scatter_body(mval_hbm, pos_hbm, idx_hbm, w16_hbm, val_hbm,
                     xv_out, lin_out,
                     pos_v, stv, idx_v, idxhi_v, w64_v, val_v, prod_v, lin_v,
                     sem_v, sem_g):
    wid = lax.axis_index("s") * NC + lax.axis_index("c")
    iota = lax.iota(jnp.int32, 16)
    fiota = iota.astype(jnp.float32)

    pltpu.sync_copy(pos_hbm.at[pl.ds(wid * NCH_B, NCH_B)], pos_v)

    def block(b, carry):
        base = wid * OUTW + b * BLK * 128
        pltpu.sync_copy(mval_hbm.at[pl.ds(base, BLK * 128)], stv)

        def fire(i, c):
            off = pl.multiple_of(i * 128, 128)
            pltpu.async_copy(stv.at[pl.ds(off, 128)],
                             xv_out.at[pos_v.at[b * BLK + i]], sem_v)
            return c

        def drain(i, c):
            pltpu.make_async_copy(stv.at[pl.ds(0, 128)],
                                  xv_out.at[pl.ds(0, 128)], sem_v).wait()
            return c

        lax.fori_loop(0, BLK, fire, 0)
        lax.fori_loop(0, BLK, drain, 0)
        return carry

    lax.fori_loop(0, NBLK, block, 0)

    cbase = wid * CHUNK_W
    pltpu.sync_copy(idx_hbm.at[pl.ds(cbase, CHUNK_W)], idx_v)
    pltpu.sync_copy(val_hbm.at[pl.ds(cbase, CHUNK_W)], val_v)

    def prep(j, carry):
        off = pl.multiple_of(j * 16, 16)
        idxhi_v[pl.ds(off, 16)] = lax.shift_right_logical(idx_v[pl.ds(off, 16)], 4)
        return carry

    lax.fori_loop(0, CHUNK_W // 16, prep, 0)

    def gchunk(j, carry):
        off = pl.multiple_of(j * 128, 128)
        pltpu.async_copy(w16_hbm.at[idxhi_v.at[pl.ds(off, 128)]],
                         w64_v.at[pl.ds(off, 128)], sem_g).wait()
        return carry

    lax.fori_loop(0, CHUNK_W // 128, gchunk, 0)

    def wmul(j, carry):
        off = pl.multiple_of(j * 16, 16)
        lanes = idx_v[pl.ds(off, 16)] & _splat(15)
        wv = plsc.load_gather(w64_v, [_splat(j * 16) + iota, lanes])
        prod_v[pl.ds(off, 16)] = wv * val_v[pl.ds(off, 16)]
        return carry

    lax.fori_loop(0, CHUNK_W // 16, wmul, 0)

    tailmask = (fiota < 10.0).astype(jnp.float32)
    lane0 = iota == 0

    def rowsum2(r, carry):
        s1 = prod_v[pl.ds(r * F, 16)]
        s2 = prod_v[pl.ds(r * F + 16, 16)] * tailmask
        tot = jnp.sum(s1 + s2, axis=0)
        plsc.store_scatter(lin_v, [_splat(r)],
                           jnp.full((16,), tot, jnp.float32), mask=lane0)
        return carry

    lax.fori_loop(0, CHUNK_W // F, rowsum2, 0)
    pltpu.sync_copy(lin_v, lin_out.at[pl.ds(wid * (CHUNK_W // F), CHUNK_W // F)])


@functools.cache
def _sc_scatter():
    return pl.kernel(
        _sc_scatter_body,
        out_type=[
            jax.ShapeDtypeStruct((BF + DUMPN, 16), jnp.float32),
            jax.ShapeDtypeStruct((B,), jnp.float32),
        ],
        mesh=plsc.VectorSubcoreMesh(core_axis_name="c", subcore_axis_name="s"),
        compiler_params=pltpu.CompilerParams(use_tc_tiling_on_sc=False,
                                             needs_layout_passes=False),
        scratch_types=[
            pltpu.VMEM((NCH_B, 128), jnp.int32),
            pltpu.VMEM((BLK * 128, 16), jnp.float32),
            pltpu.VMEM((CHUNK_W,), jnp.int32),
            pltpu.VMEM((CHUNK_W,), jnp.int32),
            pltpu.VMEM((CHUNK_W, 16), jnp.float32),
            pltpu.VMEM((CHUNK_W,), jnp.float32),
            pltpu.VMEM((CHUNK_W + 16,), jnp.float32),
            pltpu.VMEM((CHUNK_W // F,), jnp.float32),
            pltpu.SemaphoreType.DMA,
            pltpu.SemaphoreType.DMA,
        ],
    )


FDP = 512


def _sc_retile_body(in_hbm, out_hbm, buf1d, tiles, sem_o):
    wid = lax.axis_index("s") * NC + lax.axis_index("c")

    def group(g, carry):
        row0 = wid * 128 + g * 8
        pltpu.sync_copy(in_hbm.at[pl.ds(pl.multiple_of(row0 * FD, 8), 8 * FD)],
                        buf1d)
        for j in range(4):
            for r in range(8):
                for c in range(8):
                    src = r * FD + j * 128 + c * 16
                    if src + 16 <= (r + 1) * FD:
                        tiles[j * 8 + r, pl.ds(c * 16, 16)] = \
                            buf1d[pl.ds(src, 16)]
                    else:
                        tiles[j * 8 + r, pl.ds(c * 16, 16)] = \
                            jnp.zeros((16,), jnp.float32)
        for j in range(4):
            pltpu.async_copy(
                tiles.at[pl.ds(j * 8, 8), :],
                out_hbm.at[pl.ds(pl.multiple_of(row0, 8), 8),
                           pl.ds(j * 128, 128)], sem_o)
        for j in range(4):
            pltpu.make_async_copy(tiles.at[pl.ds(0, 8), :],
                                  out_hbm.at[pl.ds(0, 8), pl.ds(0, 128)],
                                  sem_o).wait()
        return carry

    lax.fori_loop(0, 16, group, 0)


@functools.cache
def _sc_retile():
    return pl.kernel(
        _sc_retile_body,
        out_type=[jax.ShapeDtypeStruct((B, FDP), jnp.float32)],
        mesh=plsc.VectorSubcoreMesh(core_axis_name="c", subcore_axis_name="s"),
        compiler_params=pltpu.CompilerParams(needs_layout_passes=False),
        scratch_types=[
            pltpu.VMEM((8 * FD,), jnp.float32),
            pltpu.VMEM((32, 128), jnp.float32),
            pltpu.SemaphoreType.DMA,
        ],
    )


BB = 512
GRID = B // BB


def _tc_body(xv_ref, fv_ref, lin_ref, w0_ref, b0_ref, w1_ref, b1_ref,
             w2_ref, b2_ref, w3t_ref, k_ref, e_ref, sc_ref, out_ref):
    fv = fv_ref[...]
    val_exp = jnp.dot(fv, e_ref[...], preferred_element_type=jnp.float32)
    xv = xv_ref[...] * val_exp
    h = jnp.maximum(jnp.dot(xv, w0_ref[...], preferred_element_type=jnp.float32)
                    + b0_ref[...], 0.0)
    h = jnp.maximum(jnp.dot(h, w1_ref[...], preferred_element_type=jnp.float32)
                    + b1_ref[...], 0.0)
    h = jnp.maximum(jnp.dot(h, w2_ref[...], preferred_element_type=jnp.float32)
                    + b2_ref[...], 0.0)
    deep = jnp.sum(h * w3t_ref[...], axis=1, keepdims=True)
    z = jnp.dot(xv, k_ref[...], preferred_element_type=jnp.float32)
    fm = 0.5 * jnp.sum(xv * z, axis=1, keepdims=True)
    out_ref[...] = (lin_ref[...] + fm + deep) * sc_ref[0] + sc_ref[1]


_tc_call = pl.pallas_call(
    _tc_body,
    grid=(GRID,),
    in_specs=[
        pl.BlockSpec((BB, FDP), lambda i: (i, 0)),
        pl.BlockSpec((BB, F), lambda i: (i, 0)),
        pl.BlockSpec((BB, 1), lambda i: (i, 0)),
        pl.BlockSpec((FDP, MLP_W), lambda i: (0, 0)),
        pl.BlockSpec((1, MLP_W), lambda i: (0, 0)),
        pl.BlockSpec((MLP_W, MLP_W), lambda i: (0, 0)),
        pl.BlockSpec((1, MLP_W), lambda i: (0, 0)),
        pl.BlockSpec((MLP_W, MLP_W), lambda i: (0, 0)),
        pl.BlockSpec((1, MLP_W), lambda i: (0, 0)),
        pl.BlockSpec((1, MLP_W), lambda i: (0, 0)),
        pl.BlockSpec((FDP, FDP), lambda i: (0, 0)),
        pl.BlockSpec((F, FDP), lambda i: (0, 0)),
        pl.BlockSpec(memory_space=pltpu.SMEM),
    ],
    out_specs=pl.BlockSpec((BB, 1), lambda i: (i, 0)),
    out_shape=jax.ShapeDtypeStruct((B, 1), jnp.float32),
)

_E = np.zeros((F, FDP), dtype=np.float32)
_E[:, :FD] = np.kron(np.eye(F, dtype=np.float32), np.ones((1, D), dtype=np.float32))


def kernel(feat_index, feat_value, w_table, v_table, W0, b0, W1, b1, W2, b2,
           W3, b3, mask, bn_gamma, bn_beta, W_out, b_out):
    idx_flat = feat_index.reshape(BF)
    vt = v_table.T
    w16 = w_table.T.reshape(V // 16, 16)
    val_flat = feat_value.reshape(BF)
    mval, pos = _sc_stream()(idx_flat, vt)
    xv_all, lin_sc = _sc_scatter()(
        mval.reshape(CAP_TOT, 16), pos.reshape(NW * NCH_B, 128),
        idx_flat, w16, val_flat)
    (xv2,) = _sc_retile()(xv_all.reshape((BF + DUMPN) * 16))
    lin2 = lin_sc.reshape(B, 1)

    coef = (mask[0] * bn_gamma) * (1.0 / np.sqrt(1.0 + 1e-3))
    M = jnp.zeros((F, F), jnp.float32).at[_PAIR_ROWS, _PAIR_COLS].set(coef)
    M = M + M.T
    K = jnp.pad(jnp.kron(M, jnp.eye(D, dtype=jnp.float32)),
                ((0, FDP - FD), (0, FDP - FD)))
    scale = W_out[0, 0]
    shift = b_out[0] + scale * (b3[0] + jnp.sum(mask[0] * bn_beta))
    sc = jnp.stack([scale, shift])

    W0p = jnp.concatenate([W0, jnp.zeros((FDP - FD, MLP_W), jnp.float32)])
    return _tc_call(
        xv2, feat_value, lin2, W0p, b0.reshape(1, MLP_W), W1,
        b1.reshape(1, MLP_W), W2, b2.reshape(1, MLP_W), W3.reshape(1, MLP_W),
        K, _E, sc)


# speedup vs baseline: 1.2206x; 1.0249x over previous
import functools
from itertools import combinations

import jax
import jax.numpy as jnp
import numpy as np
from jax import lax
from jax.experimental import pallas as pl
from jax.experimental.pallas import tpu as pltpu
from jax.experimental.pallas import tpu_sc as plsc

B, F, V, D = 4096, 26, 1000000, 16
FD = F * D
BF = B * F
MLP_W = 256
NUM_PAIRS = F * (F - 1) // 2

NC, NS = 2, 16
NW = NC * NS

NT_TILES = (V + 127) // 128
TPW = 244
NTS = 28
NBINS = 9
CAPB = 640
OUTW = NBINS * CAPB
CAP_TOT = NW * OUTW
M_CAP = 4608
SCCHUNK = 2048
NSCCH = BF // SCCHUNK
NCH_B = OUTW // 128
DUMPN = 104
DUMP0 = BF
CHUNK_W = BF // NW

_PAIR_ROWS, _PAIR_COLS = map(np.array, zip(*combinations(range(F), 2)))
_IOTA16 = np.arange(16, dtype=np.int32)


def _splat(x):
    return jnp.full((16,), x, dtype=jnp.int32)


def _sc_stream_body(idx_hbm, vt_hbm, mval_out, pos_out,
                    idx_st, r_list, p_list, rbin, pbin, slab,
                    vstage, sem_t):
    wid = lax.axis_index("s") * NC + lax.axis_index("c")
    ja = wid * TPW
    jb = jnp.where(wid == NW - 1, NT_TILES, ja + TPW)
    iota = lax.iota(jnp.int32, 16)

    def scan_chunk(c, cnt):
        pltpu.sync_copy(idx_hbm.at[pl.ds(c * SCCHUNK, SCCHUNK)], idx_st)

        def scan_vec(i, cnt):
            rvec = idx_st[pl.ds(i * 16, 16)]
            j7 = lax.shift_right_logical(rvec, 7)
            m = (j7 >= _splat(ja)) & (j7 < _splat(jb))
            posvec = _splat(c * SCCHUNK + i * 16) + iota
            inc = plsc.cumsum(m.astype(jnp.int32))
            dest = jnp.minimum(cnt + inc - 1, _splat(M_CAP - 1))
            plsc.store_scatter(r_list, [dest], rvec, mask=m)
            plsc.store_scatter(p_list, [dest], posvec, mask=m)
            return cnt + plsc.all_reduce_population_count(m)

        return lax.fori_loop(0, SCCHUNK // 16, scan_vec, cnt, unroll=4)

    cnt_vec = lax.fori_loop(0, NSCCH, scan_chunk, jnp.zeros((16,), jnp.int32))
    cnt = jnp.max(cnt_vec, axis=0)

    def fill_dump(i, carry):
        dv = _splat(DUMP0) + iota + _splat(16 * (i % 6))
        pbin[pl.ds(i * 16, 16)] = dv
        return carry

    lax.fori_loop(0, NBINS * CAPB // 16, fill_dump, 0)

    def bin_vec(i, cnts):
        rvec = r_list[pl.ds(i * 16, 16)]
        pvec = p_list[pl.ds(i * 16, 16)]
        valid = (_splat(i * 16) + iota) < cnt_vec
        bb = (lax.shift_right_logical(rvec, 7) - _splat(ja)) // NTS
        new = []
        for k in range(NBINS):
            ck = cnts[k]
            mk = valid & (bb == _splat(k))
            inc = plsc.cumsum(mk.astype(jnp.int32))
            dest = jnp.minimum(ck + inc - 1, _splat(CAPB - 1)) + _splat(k * CAPB)
            plsc.store_scatter(rbin, [dest], rvec, mask=mk)
            plsc.store_scatter(pbin, [dest], pvec, mask=mk)
            new.append(ck + plsc.all_reduce_population_count(mk))
        return tuple(new)

    cnts_vec = lax.fori_loop(0, (cnt + 15) // 16, bin_vec,
                             tuple(jnp.zeros((16,), jnp.int32)
                                   for _ in range(NBINS)))
    cnts = [jnp.max(cv, axis=0) for cv in cnts_vec]

    for k in range(NBINS):
        j0 = ja + k * NTS
        ntk = jnp.maximum(jnp.minimum(jb - j0, NTS), 0)

        def fire(i, carry, j0=j0):
            jj = j0 + lax.shift_right_logical(i, 1)
            t = i & 1
            row = (lax.shift_right_logical(i, 1) * 16 + t * 8)
            pltpu.async_copy(
                vt_hbm.at[pl.ds(t * 8, 8), pl.ds(pl.multiple_of(jj * 128, 128), 128)],
                slab.at[pl.ds(pl.multiple_of(row, 8), 8), :], sem_t)
            return carry

        def drain(i, carry):
            pltpu.make_async_copy(
                vt_hbm.at[pl.ds(0, 8), pl.ds(0, 128)],
                slab.at[pl.ds(0, 8), :], sem_t).wait()
            return carry

        lax.fori_loop(0, 2 * ntk, fire, 0)
        lax.fori_loop(0, 2 * ntk, drain, 0)

        def extract(m, carry, k=k, j0=j0):
            rsp = plsc.load_gather(rbin, [_splat(k * CAPB + m)])
            jloc = lax.shift_right_logical(rsp, 7) - _splat(j0)
            rows = jloc * 16 + iota
            cols = rsp & _splat(127)
            v16 = plsc.load_gather(slab, [rows, cols])
            vstage[pl.ds(m * 16, 16)] = v16
            return carry

        lax.fori_loop(0, cnts[k], extract, 0)
        base = (wid * NBINS + k) * CAPB
        pltpu.sync_copy(vstage, mval_out.at[pl.ds(base * 16, CAPB * 16)])
        pltpu.sync_copy(pbin.at[pl.ds(k * CAPB, CAPB)],
                        pos_out.at[pl.ds(base, CAPB)])


@functools.cache
def _sc_stream():
    return pl.kernel(
        _sc_stream_body,
        out_type=[
            jax.ShapeDtypeStruct((CAP_TOT * 16,), jnp.float32),
            jax.ShapeDtypeStruct((CAP_TOT,), jnp.int32),
        ],
        mesh=plsc.VectorSubcoreMesh(core_axis_name="c", subcore_axis_name="s"),
        compiler_params=pltpu.CompilerParams(needs_layout_passes=False),
        scratch_types=[
            pltpu.VMEM((SCCHUNK,), jnp.int32),
            pltpu.VMEM((M_CAP,), jnp.int32),
            pltpu.VMEM((M_CAP,), jnp.int32),
            pltpu.VMEM((NBINS * CAPB,), jnp.int32),
            pltpu.VMEM((NBINS * CAPB,), jnp.int32),
            pltpu.VMEM((NTS * 16, 128), jnp.float32),
            pltpu.VMEM((CAPB * 16,), jnp.float32),
            pltpu.SemaphoreType.DMA,
        ],
    )


BLK = 15
NBLK = NCH_B // BLK


def _sc_scatter_body(mval_hbm, pos_hbm, idx_hbm, w16_hbm, val_hbm,
                     xv_out, lin_out,
                     pos_v, stv, idx_v, idxhi_v, w64_v, val_v, prod_v, lin_v,
                     sem_v, sem_g):
    wid = lax.axis_index("s") * NC + lax.axis_index("c")
    iota = lax.iota(jnp.int32, 16)
    fiota = iota.astype(jnp.float32)

    pltpu.sync_copy(pos_hbm.at[pl.ds(wid * NCH_B, NCH_B)], pos_v)

    def block(b, carry):
        base = wid * OUTW + b * BLK * 128
        pltpu.sync_copy(mval_hbm.at[pl.ds(base, BLK * 128)], stv)

        def fire(i, c):
            off = pl.multiple_of(i * 128, 128)
            pltpu.async_copy(stv.at[pl.ds(off, 128)],
                             xv_out.at[pos_v.at[b * BLK + i]], sem_v)
            return c

        def drain(i, c):
            pltpu.make_async_copy(stv.at[pl.ds(0, 128)],
                                  xv_out.at[pl.ds(0, 128)], sem_v).wait()
            return c

        lax.fori_loop(0, BLK, fire, 0)
        lax.fori_loop(0, BLK, drain, 0)
        return carry

    lax.fori_loop(0, NBLK, block, 0)

    cbase = wid * CHUNK_W
    pltpu.sync_copy(idx_hbm.at[pl.ds(cbase, CHUNK_W)], idx_v)
    pltpu.sync_copy(val_hbm.at[pl.ds(cbase, CHUNK_W)], val_v)

    def prep(j, carry):
        off = pl.multiple_of(j * 16, 16)
        idxhi_v[pl.ds(off, 16)] = lax.shift_right_logical(idx_v[pl.ds(off, 16)], 4)
        return carry

    lax.fori_loop(0, CHUNK_W // 16, prep, 0, unroll=4)

    def gfire(j, carry):
        off = pl.multiple_of(j * 128, 128)
        pltpu.async_copy(w16_hbm.at[idxhi_v.at[pl.ds(off, 128)]],
                         w64_v.at[pl.ds(off, 128)], sem_g)
        return carry

    def gdrain(j, carry):
        pltpu.make_async_copy(w16_hbm.at[idxhi_v.at[pl.ds(0, 128)]],
                              w64_v.at[pl.ds(0, 128)], sem_g).wait()
        return carry

    lax.fori_loop(0, CHUNK_W // 128, gfire, 0)
    lax.fori_loop(0, CHUNK_W // 128, gdrain, 0)

    def wmul(j, carry):
        off = pl.multiple_of(j * 16, 16)
        lanes = idx_v[pl.ds(off, 16)] & _splat(15)
        wv = plsc.load_gather(w64_v, [_splat(j * 16) + iota, lanes])
        prod_v[pl.ds(off, 16)] = wv * val_v[pl.ds(off, 16)]
        return carry

    lax.fori_loop(0, CHUNK_W // 16, wmul, 0, unroll=4)

    tailmask = (fiota < 10.0).astype(jnp.float32)
    lane0 = iota == 0

    def rowsum2(r, carry):
        s1 = prod_v[pl.ds(r * F, 16)]
        s2 = prod_v[pl.ds(r * F + 16, 16)] * tailmask
        tot = jnp.sum(s1 + s2, axis=0)
        plsc.store_scatter(lin_v, [_splat(r)],
                           jnp.full((16,), tot, jnp.float32), mask=lane0)
        return carry

    lax.fori_loop(0, CHUNK_W // F, rowsum2, 0)
    pltpu.sync_copy(lin_v, lin_out.at[pl.ds(wid * (CHUNK_W // F), CHUNK_W // F)])


@functools.cache
def _sc_scatter():
    return pl.kernel(
        _sc_scatter_body,
        out_type=[
            jax.ShapeDtypeStruct((BF + DUMPN, 16), jnp.float32),
            jax.ShapeDtypeStruct((B,), jnp.float32),
        ],
        mesh=plsc.VectorSubcoreMesh(core_axis_name="c", subcore_axis_name="s"),
        compiler_params=pltpu.CompilerParams(use_tc_tiling_on_sc=False,
                                             needs_layout_passes=False),
        scratch_types=[
            pltpu.VMEM((NCH_B, 128), jnp.int32),
            pltpu.VMEM((BLK * 128, 16), jnp.float32),
            pltpu.VMEM((CHUNK_W,), jnp.int32),
            pltpu.VMEM((CHUNK_W,), jnp.int32),
            pltpu.VMEM((CHUNK_W, 16), jnp.float32),
            pltpu.VMEM((CHUNK_W,), jnp.float32),
            pltpu.VMEM((CHUNK_W + 16,), jnp.float32),
            pltpu.VMEM((CHUNK_W // F,), jnp.float32),
            pltpu.SemaphoreType.DMA,
            pltpu.SemaphoreType.DMA,
        ],
    )


FDP = 512


def _sc_retile_body(in_hbm, out_hbm, buf1d, tiles, sem_o):
    wid = lax.axis_index("s") * NC + lax.axis_index("c")

    def group(g, carry):
        row0 = wid * 128 + g * 8

        @pl.when(g > 0)
        def _():
            for j in range(4):
                pltpu.make_async_copy(tiles.at[pl.ds(0, 8), :],
                                      out_hbm.at[pl.ds(0, 8), pl.ds(0, 128)],
                                      sem_o).wait()

        pltpu.sync_copy(in_hbm.at[pl.ds(pl.multiple_of(row0 * FD, 8), 8 * FD)],
                        buf1d)
        for j in range(4):
            for r in range(8):
                for c in range(8):
                    src = r * FD + j * 128 + c * 16
                    if src + 16 <= (r + 1) * FD:
                        tiles[j * 8 + r, pl.ds(c * 16, 16)] = \
                            buf1d[pl.ds(src, 16)]
                    else:
                        tiles[j * 8 + r, pl.ds(c * 16, 16)] = \
                            jnp.zeros((16,), jnp.float32)
        for j in range(4):
            pltpu.async_copy(
                tiles.at[pl.ds(j * 8, 8), :],
                out_hbm.at[pl.ds(pl.multiple_of(row0, 8), 8),
                           pl.ds(j * 128, 128)], sem_o)
        return carry

    lax.fori_loop(0, 16, group, 0)
    for j in range(4):
        pltpu.make_async_copy(tiles.at[pl.ds(0, 8), :],
                              out_hbm.at[pl.ds(0, 8), pl.ds(0, 128)],
                              sem_o).wait()


@functools.cache
def _sc_retile():
    return pl.kernel(
        _sc_retile_body,
        out_type=[jax.ShapeDtypeStruct((B, FDP), jnp.float32)],
        mesh=plsc.VectorSubcoreMesh(core_axis_name="c", subcore_axis_name="s"),
        compiler_params=pltpu.CompilerParams(needs_layout_passes=False),
        scratch_types=[
            pltpu.VMEM((8 * FD,), jnp.float32),
            pltpu.VMEM((32, 128), jnp.float32),
            pltpu.SemaphoreType.DMA,
        ],
    )


BB = 512
GRID = B // BB


def _tc_body(xv_ref, fv_ref, lin_ref, w0_ref, b0_ref, w1_ref, b1_ref,
             w2_ref, b2_ref, w3t_ref, k_ref, e_ref, sc_ref, out_ref):
    fv = fv_ref[...]
    val_exp = jnp.dot(fv, e_ref[...], preferred_element_type=jnp.float32)
    xv = xv_ref[...] * val_exp
    h = jnp.maximum(jnp.dot(xv, w0_ref[...], preferred_element_type=jnp.float32)
                    + b0_ref[...], 0.0)
    h = jnp.maximum(jnp.dot(h, w1_ref[...], preferred_element_type=jnp.float32)
                    + b1_ref[...], 0.0)
    h = jnp.maximum(jnp.dot(h, w2_ref[...], preferred_element_type=jnp.float32)
                    + b2_ref[...], 0.0)
    deep = jnp.sum(h * w3t_ref[...], axis=1, keepdims=True)
    z = jnp.dot(xv, k_ref[...], preferred_element_type=jnp.float32)
    fm = 0.5 * jnp.sum(xv * z, axis=1, keepdims=True)
    out_ref[...] = (lin_ref[...] + fm + deep) * sc_ref[0] + sc_ref[1]


_tc_call = pl.pallas_call(
    _tc_body,
    grid=(GRID,),
    in_specs=[
        pl.BlockSpec((BB, FDP), lambda i: (i, 0)),
        pl.BlockSpec((BB, F), lambda i: (i, 0)),
        pl.BlockSpec((BB, 1), lambda i: (i, 0)),
        pl.BlockSpec((FDP, MLP_W), lambda i: (0, 0)),
        pl.BlockSpec((1, MLP_W), lambda i: (0, 0)),
        pl.BlockSpec((MLP_W, MLP_W), lambda i: (0, 0)),
        pl.BlockSpec((1, MLP_W), lambda i: (0, 0)),
        pl.BlockSpec((MLP_W, MLP_W), lambda i: (0, 0)),
        pl.BlockSpec((1, MLP_W), lambda i: (0, 0)),
        pl.BlockSpec((1, MLP_W), lambda i: (0, 0)),
        pl.BlockSpec((FDP, FDP), lambda i: (0, 0)),
        pl.BlockSpec((F, FDP), lambda i: (0, 0)),
        pl.BlockSpec(memory_space=pltpu.SMEM),
    ],
    out_specs=pl.BlockSpec((BB, 1), lambda i: (i, 0)),
    out_shape=jax.ShapeDtypeStruct((B, 1), jnp.float32),
)

_E = np.zeros((F, FDP), dtype=np.float32)
_E[:, :FD] = np.kron(np.eye(F, dtype=np.float32), np.ones((1, D), dtype=np.float32))


def kernel(feat_index, feat_value, w_table, v_table, W0, b0, W1, b1, W2, b2,
           W3, b3, mask, bn_gamma, bn_beta, W_out, b_out):
    idx_flat = feat_index.reshape(BF)
    vt = v_table.T
    w16 = w_table.T.reshape(V // 16, 16)
    val_flat = feat_value.reshape(BF)
    mval, pos = _sc_stream()(idx_flat, vt)
    xv_all, lin_sc = _sc_scatter()(
        mval.reshape(CAP_TOT, 16), pos.reshape(NW * NCH_B, 128),
        idx_flat, w16, val_flat)
    (xv2,) = _sc_retile()(xv_all.reshape((BF + DUMPN) * 16))
    lin2 = lin_sc.reshape(B, 1)

    coef = (mask[0] * bn_gamma) * (1.0 / np.sqrt(1.0 + 1e-3))
    M = jnp.zeros((F, F), jnp.float32).at[_PAIR_ROWS, _PAIR_COLS].set(coef)
    M = M + M.T
    K = jnp.pad(jnp.kron(M, jnp.eye(D, dtype=jnp.float32)),
                ((0, FDP - FD), (0, FDP - FD)))
    scale = W_out[0, 0]
    shift = b_out[0] + scale * (b3[0] + jnp.sum(mask[0] * bn_beta))
    sc = jnp.stack([scale, shift])

    W0p = jnp.concatenate([W0, jnp.zeros((FDP - FD, MLP_W), jnp.float32)])
    return _tc_call(
        xv2, feat_value, lin2, W0p, b0.reshape(1, MLP_W), W1,
        b1.reshape(1, MLP_W), W2, b2.reshape(1, MLP_W), W3.reshape(1, MLP_W),
        K, _E, sc)


# speedup vs baseline: 1.3759x; 1.1272x over previous
import functools
from itertools import combinations

import jax
import jax.numpy as jnp
import numpy as np
from jax import lax
from jax.experimental import pallas as pl
from jax.experimental.pallas import tpu as pltpu
from jax.experimental.pallas import tpu_sc as plsc

B, F, V, D = 4096, 26, 1000000, 16
FD = F * D
BF = B * F
MLP_W = 256
NUM_PAIRS = F * (F - 1) // 2

NC, NS = 2, 16
NW = NC * NS

NT_TILES = (V + 127) // 128
TPW = 244
NTS = 16
NBINS = 16
CAPB = 384
OUTW = NBINS * CAPB
CAP_TOT = NW * OUTW
M_CAP = 4608
SCCHUNK = 2048
NSCCH = BF // SCCHUNK
NCH_B = OUTW // 128
DUMPN = 104
DUMP0 = BF
CHUNK_W = BF // NW

_PAIR_ROWS, _PAIR_COLS = map(np.array, zip(*combinations(range(F), 2)))
_IOTA16 = np.arange(16, dtype=np.int32)


def _splat(x):
    return jnp.full((16,), x, dtype=jnp.int32)


def _sc_stream_body(idx_hbm, vt_hbm, mval_out, pos_out,
                    idx_st, idx_st2, r_list, p_list, rbin, pbin, slab,
                    vstage, sem_t, sem_t2, sem_i, sem_i2):
    wid = lax.axis_index("s") * NC + lax.axis_index("c")
    ja = wid * TPW
    jb = jnp.where(wid == NW - 1, NT_TILES, ja + TPW)
    iota = lax.iota(jnp.int32, 16)

    def scan_buf(buf, base, cnt):
        def scan_vec(i, cnt):
            rvec = buf[pl.ds(i * 16, 16)]
            j7 = lax.shift_right_logical(rvec, 7)
            m = (j7 >= _splat(ja)) & (j7 < _splat(jb))
            posvec = base + _splat(i * 16) + iota
            inc = plsc.cumsum(m.astype(jnp.int32))
            dest = jnp.minimum(cnt + inc - 1, _splat(M_CAP - 1))
            plsc.store_scatter(r_list, [dest], rvec, mask=m)
            plsc.store_scatter(p_list, [dest], posvec, mask=m)
            return cnt + plsc.all_reduce_population_count(m)

        return lax.fori_loop(0, SCCHUNK // 16, scan_vec, cnt, unroll=4)

    pltpu.sync_copy(idx_hbm.at[pl.ds(0, SCCHUNK)], idx_st)

    def scan_pair(c, cnt):
        @pl.when(c > 0)
        def _():
            pltpu.make_async_copy(idx_hbm.at[pl.ds(0, SCCHUNK)],
                                  idx_st, sem_i).wait()
        pltpu.async_copy(
            idx_hbm.at[pl.ds((2 * c + 1) * SCCHUNK, SCCHUNK)], idx_st2, sem_i2)
        cnt = scan_buf(idx_st, _splat(2 * c * SCCHUNK), cnt)

        @pl.when(c < NSCCH // 2 - 1)
        def _():
            pltpu.async_copy(
                idx_hbm.at[pl.ds((2 * c + 2) * SCCHUNK, SCCHUNK)], idx_st, sem_i)
        pltpu.make_async_copy(idx_hbm.at[pl.ds(0, SCCHUNK)],
                              idx_st2, sem_i2).wait()
        cnt = scan_buf(idx_st2, _splat((2 * c + 1) * SCCHUNK), cnt)
        return cnt

    cnt_vec = lax.fori_loop(0, NSCCH // 2, scan_pair, jnp.zeros((16,), jnp.int32))
    cnt = jnp.max(cnt_vec, axis=0)

    def fill_dump(i, carry):
        dv = _splat(DUMP0) + iota + _splat(16 * (i % 6))
        pbin[pl.ds(i * 16, 16)] = dv
        return carry

    lax.fori_loop(0, NBINS * CAPB // 16, fill_dump, 0)

    def bin_vec(i, cnts):
        rvec = r_list[pl.ds(i * 16, 16)]
        pvec = p_list[pl.ds(i * 16, 16)]
        valid = (_splat(i * 16) + iota) < cnt_vec
        bb = (lax.shift_right_logical(rvec, 7) - _splat(ja)) // NTS
        new = []
        for k in range(NBINS):
            ck = cnts[k]
            mk = valid & (bb == _splat(k))
            inc = plsc.cumsum(mk.astype(jnp.int32))
            dest = jnp.minimum(ck + inc - 1, _splat(CAPB - 1)) + _splat(k * CAPB)
            plsc.store_scatter(rbin, [dest], rvec, mask=mk)
            plsc.store_scatter(pbin, [dest], pvec, mask=mk)
            new.append(ck + plsc.all_reduce_population_count(mk))
        return tuple(new)

    cnts_vec = lax.fori_loop(0, (cnt + 15) // 16, bin_vec,
                             tuple(jnp.zeros((16,), jnp.int32)
                                   for _ in range(NBINS)))
    cnts = [jnp.max(cv, axis=0) for cv in cnts_vec]

    def fire(k, j0, half, sem):
        ntk = jnp.maximum(jnp.minimum(jb - j0, NTS), 0)

        def fire1(i, carry):
            jj = j0 + lax.shift_right_logical(i, 1)
            t = i & 1
            row = half * (NTS * 16) + (lax.shift_right_logical(i, 1) * 16 + t * 8)
            pltpu.async_copy(
                vt_hbm.at[pl.ds(t * 8, 8), pl.ds(pl.multiple_of(jj * 128, 128), 128)],
                slab.at[pl.ds(pl.multiple_of(row, 8), 8), :], sem)
            return carry

        lax.fori_loop(0, 2 * ntk, fire1, 0)
        return ntk

    def drain_n(ntk, sem):
        def drain1(i, carry):
            pltpu.make_async_copy(
                vt_hbm.at[pl.ds(0, 8), pl.ds(0, 128)],
                slab.at[pl.ds(0, 8), :], sem).wait()
            return carry

        lax.fori_loop(0, 2 * ntk, drain1, 0)

    ntks = [None] * NBINS
    ntks[0] = fire(0, ja, 0, sem_t)
    for k in range(NBINS):
        half = k % 2
        sem = sem_t if half == 0 else sem_t2
        if k + 1 < NBINS:
            nsem = sem_t2 if half == 0 else sem_t
            ntks[k + 1] = fire(k + 1, ja + (k + 1) * NTS, 1 - half, nsem)
        drain_n(ntks[k], sem)

        def extract(m, carry, k=k, half=half):
            rsp = plsc.load_gather(rbin, [_splat(k * CAPB + m)])
            jloc = lax.shift_right_logical(rsp, 7) - _splat(ja + k * NTS)
            rows = _splat(half * (NTS * 16)) + jloc * 16 + iota
            cols = rsp & _splat(127)
            v16 = plsc.load_gather(slab, [rows, cols])
            vstage[pl.ds(m * 16, 16)] = v16
            return carry

        lax.fori_loop(0, cnts[k], extract, 0)
        base = (wid * NBINS + k) * CAPB
        pltpu.sync_copy(vstage, mval_out.at[pl.ds(base * 16, CAPB * 16)])
        pltpu.sync_copy(pbin.at[pl.ds(k * CAPB, CAPB)],
                        pos_out.at[pl.ds(base, CAPB)])


@functools.cache
def _sc_stream():
    return pl.kernel(
        _sc_stream_body,
        out_type=[
            jax.ShapeDtypeStruct((CAP_TOT * 16,), jnp.float32),
            jax.ShapeDtypeStruct((CAP_TOT,), jnp.int32),
        ],
        mesh=plsc.VectorSubcoreMesh(core_axis_name="c", subcore_axis_name="s"),
        compiler_params=pltpu.CompilerParams(needs_layout_passes=False),
        scratch_types=[
            pltpu.VMEM((SCCHUNK,), jnp.int32),
            pltpu.VMEM((SCCHUNK,), jnp.int32),
            pltpu.VMEM((M_CAP,), jnp.int32),
            pltpu.VMEM((M_CAP,), jnp.int32),
            pltpu.VMEM((NBINS * CAPB,), jnp.int32),
            pltpu.VMEM((NBINS * CAPB,), jnp.int32),
            pltpu.VMEM((2 * NTS * 16, 128), jnp.float32),
            pltpu.VMEM((CAPB * 16,), jnp.float32),
            pltpu.SemaphoreType.DMA,
            pltpu.SemaphoreType.DMA,
            pltpu.SemaphoreType.DMA,
            pltpu.SemaphoreType.DMA,
        ],
    )


BLK = 12
NBLK = NCH_B // BLK


def _sc_scatter_body(mval_hbm, pos_hbm, idx_hbm, w16_hbm, val_hbm,
                     xv_out, lin_out,
                     pos_v, stv, idx_v, idxhi_v, w64_v, val_v, prod_v, lin_v,
                     sem_v, sem_g):
    wid = lax.axis_index("s") * NC + lax.axis_index("c")
    iota = lax.iota(jnp.int32, 16)
    fiota = iota.astype(jnp.float32)

    pltpu.sync_copy(pos_hbm.at[pl.ds(wid * NCH_B, NCH_B)], pos_v)

    def block(b, carry):
        base = wid * OUTW + b * BLK * 128
        pltpu.sync_copy(mval_hbm.at[pl.ds(base, BLK * 128)], stv)

        def fire(i, c):
            off = pl.multiple_of(i * 128, 128)
            pltpu.async_copy(stv.at[pl.ds(off, 128)],
                             xv_out.at[pos_v.at[b * BLK + i]], sem_v)
            return c

        def drain(i, c):
            pltpu.make_async_copy(stv.at[pl.ds(0, 128)],
                                  xv_out.at[pl.ds(0, 128)], sem_v).wait()
            return c

        lax.fori_loop(0, BLK, fire, 0)
        lax.fori_loop(0, BLK, drain, 0)
        return carry

    lax.fori_loop(0, NBLK, block, 0)

    cbase = wid * CHUNK_W
    pltpu.sync_copy(idx_hbm.at[pl.ds(cbase, CHUNK_W)], idx_v)
    pltpu.sync_copy(val_hbm.at[pl.ds(cbase, CHUNK_W)], val_v)

    def prep(j, carry):
        off = pl.multiple_of(j * 16, 16)
        idxhi_v[pl.ds(off, 16)] = lax.shift_right_logical(idx_v[pl.ds(off, 16)], 4)
        return carry

    lax.fori_loop(0, CHUNK_W // 16, prep, 0, unroll=4)

    def gfire(j, carry):
        off = pl.multiple_of(j * 128, 128)
        pltpu.async_copy(w16_hbm.at[idxhi_v.at[pl.ds(off, 128)]],
                         w64_v.at[pl.ds(off, 128)], sem_g)
        return carry

    def gdrain(j, carry):
        pltpu.make_async_copy(w16_hbm.at[idxhi_v.at[pl.ds(0, 128)]],
                              w64_v.at[pl.ds(0, 128)], sem_g).wait()
        return carry

    lax.fori_loop(0, CHUNK_W // 128, gfire, 0)
    lax.fori_loop(0, CHUNK_W // 128, gdrain, 0)

    def wmul(j, carry):
        off = pl.multiple_of(j * 16, 16)
        lanes = idx_v[pl.ds(off, 16)] & _splat(15)
        wv = plsc.load_gather(w64_v, [_splat(j * 16) + iota, lanes])
        prod_v[pl.ds(off, 16)] = wv * val_v[pl.ds(off, 16)]
        return carry

    lax.fori_loop(0, CHUNK_W // 16, wmul, 0, unroll=4)

    tailmask = (fiota < 10.0).astype(jnp.float32)
    lane0 = iota == 0

    def rowsum2(r, carry):
        s1 = prod_v[pl.ds(r * F, 16)]
        s2 = prod_v[pl.ds(r * F + 16, 16)] * tailmask
        tot = jnp.sum(s1 + s2, axis=0)
        plsc.store_scatter(lin_v, [_splat(r)],
                           jnp.full((16,), tot, jnp.float32), mask=lane0)
        return carry

    lax.fori_loop(0, CHUNK_W // F, rowsum2, 0)
    pltpu.sync_copy(lin_v, lin_out.at[pl.ds(wid * (CHUNK_W // F), CHUNK_W // F)])


@functools.cache
def _sc_scatter():
    return pl.kernel(
        _sc_scatter_body,
        out_type=[
            jax.ShapeDtypeStruct((BF + DUMPN, 16), jnp.float32),
            jax.ShapeDtypeStruct((B,), jnp.float32),
        ],
        mesh=plsc.VectorSubcoreMesh(core_axis_name="c", subcore_axis_name="s"),
        compiler_params=pltpu.CompilerParams(use_tc_tiling_on_sc=False,
                                             needs_layout_passes=False),
        scratch_types=[
            pltpu.VMEM((NCH_B, 128), jnp.int32),
            pltpu.VMEM((BLK * 128, 16), jnp.float32),
            pltpu.VMEM((CHUNK_W,), jnp.int32),
            pltpu.VMEM((CHUNK_W,), jnp.int32),
            pltpu.VMEM((CHUNK_W, 16), jnp.float32),
            pltpu.VMEM((CHUNK_W,), jnp.float32),
            pltpu.VMEM((CHUNK_W + 16,), jnp.float32),
            pltpu.VMEM((CHUNK_W // F,), jnp.float32),
            pltpu.SemaphoreType.DMA,
            pltpu.SemaphoreType.DMA,
        ],
    )


FDP = 512


def _sc_retile_body(in_hbm, out_hbm, buf1d, tiles, sem_o):
    wid = lax.axis_index("s") * NC + lax.axis_index("c")

    def group(g, carry):
        row0 = wid * 128 + g * 8

        @pl.when(g > 0)
        def _():
            for j in range(4):
                pltpu.make_async_copy(tiles.at[pl.ds(0, 8), :],
                                      out_hbm.at[pl.ds(0, 8), pl.ds(0, 128)],
                                      sem_o).wait()

        pltpu.sync_copy(in_hbm.at[pl.ds(pl.multiple_of(row0 * FD, 8), 8 * FD)],
                        buf1d)
        for j in range(4):
            for r in range(8):
                for c in range(8):
                    src = r * FD + j * 128 + c * 16
                    if src + 16 <= (r + 1) * FD:
                        tiles[j * 8 + r, pl.ds(c * 16, 16)] = \
                            buf1d[pl.ds(src, 16)]
                    else:
                        tiles[j * 8 + r, pl.ds(c * 16, 16)] = \
                            jnp.zeros((16,), jnp.float32)
        for j in range(4):
            pltpu.async_copy(
                tiles.at[pl.ds(j * 8, 8), :],
                out_hbm.at[pl.ds(pl.multiple_of(row0, 8), 8),
                           pl.ds(j * 128, 128)], sem_o)
        return carry

    lax.fori_loop(0, 16, group, 0)
    for j in range(4):
        pltpu.make_async_copy(tiles.at[pl.ds(0, 8), :],
                              out_hbm.at[pl.ds(0, 8), pl.ds(0, 128)],
                              sem_o).wait()


@functools.cache
def _sc_retile():
    return pl.kernel(
        _sc_retile_body,
        out_type=[jax.ShapeDtypeStruct((B, FDP), jnp.float32)],
        mesh=plsc.VectorSubcoreMesh(core_axis_name="c", subcore_axis_name="s"),
        compiler_params=pltpu.CompilerParams(needs_layout_passes=False),
        scratch_types=[
            pltpu.VMEM((8 * FD,), jnp.float32),
            pltpu.VMEM((32, 128), jnp.float32),
            pltpu.SemaphoreType.DMA,
        ],
    )


BB = 512
GRID = B // BB


def _tc_body(xv_ref, fv_ref, lin_ref, w0_ref, b0_ref, w1_ref, b1_ref,
             w2_ref, b2_ref, w3t_ref, k_ref, e_ref, sc_ref, out_ref):
    fv = fv_ref[...]
    val_exp = jnp.dot(fv, e_ref[...], preferred_element_type=jnp.float32)
    xv = xv_ref[...] * val_exp
    h = jnp.maximum(jnp.dot(xv, w0_ref[...], preferred_element_type=jnp.float32)
                    + b0_ref[...], 0.0)
    h = jnp.maximum(jnp.dot(h, w1_ref[...], preferred_element_type=jnp.float32)
                    + b1_ref[...], 0.0)
    h = jnp.maximum(jnp.dot(h, w2_ref[...], preferred_element_type=jnp.float32)
                    + b2_ref[...], 0.0)
    deep = jnp.sum(h * w3t_ref[...], axis=1, keepdims=True)
    z = jnp.dot(xv, k_ref[...], preferred_element_type=jnp.float32)
    fm = 0.5 * jnp.sum(xv * z, axis=1, keepdims=True)
    out_ref[...] = (lin_ref[...] + fm + deep) * sc_ref[0] + sc_ref[1]


_tc_call = pl.pallas_call(
    _tc_body,
    grid=(GRID,),
    in_specs=[
        pl.BlockSpec((BB, FDP), lambda i: (i, 0)),
        pl.BlockSpec((BB, F), lambda i: (i, 0)),
        pl.BlockSpec((BB, 1), lambda i: (i, 0)),
        pl.BlockSpec((FDP, MLP_W), lambda i: (0, 0)),
        pl.BlockSpec((1, MLP_W), lambda i: (0, 0)),
        pl.BlockSpec((MLP_W, MLP_W), lambda i: (0, 0)),
        pl.BlockSpec((1, MLP_W), lambda i: (0, 0)),
        pl.BlockSpec((MLP_W, MLP_W), lambda i: (0, 0)),
        pl.BlockSpec((1, MLP_W), lambda i: (0, 0)),
        pl.BlockSpec((1, MLP_W), lambda i: (0, 0)),
        pl.BlockSpec((FDP, FDP), lambda i: (0, 0)),
        pl.BlockSpec((F, FDP), lambda i: (0, 0)),
        pl.BlockSpec(memory_space=pltpu.SMEM),
    ],
    out_specs=pl.BlockSpec((BB, 1), lambda i: (i, 0)),
    out_shape=jax.ShapeDtypeStruct((B, 1), jnp.float32),
)

_E = np.zeros((F, FDP), dtype=np.float32)
_E[:, :FD] = np.kron(np.eye(F, dtype=np.float32), np.ones((1, D), dtype=np.float32))


def kernel(feat_index, feat_value, w_table, v_table, W0, b0, W1, b1, W2, b2,
           W3, b3, mask, bn_gamma, bn_beta, W_out, b_out):
    idx_flat = feat_index.reshape(BF)
    vt = v_table.T
    w16 = w_table.T.reshape(V // 16, 16)
    val_flat = feat_value.reshape(BF)
    mval, pos = _sc_stream()(idx_flat, vt)
    xv_all, lin_sc = _sc_scatter()(
        mval.reshape(CAP_TOT, 16), pos.reshape(NW * NCH_B, 128),
        idx_flat, w16, val_flat)
    (xv2,) = _sc_retile()(xv_all.reshape((BF + DUMPN) * 16))
    lin2 = lin_sc.reshape(B, 1)

    coef = (mask[0] * bn_gamma) * (1.0 / np.sqrt(1.0 + 1e-3))
    M = jnp.zeros((F, F), jnp.float32).at[_PAIR_ROWS, _PAIR_COLS].set(coef)
    M = M + M.T
    K = jnp.pad(jnp.kron(M, jnp.eye(D, dtype=jnp.float32)),
                ((0, FDP - FD), (0, FDP - FD)))
    scale = W_out[0, 0]
    shift = b_out[0] + scale * (b3[0] + jnp.sum(mask[0] * bn_beta))
    sc = jnp.stack([scale, shift])

    W0p = jnp.concatenate([W0, jnp.zeros((FDP - FD, MLP_W), jnp.float32)])
    return _tc_call(
        xv2, feat_value, lin2, W0p, b0.reshape(1, MLP_W), W1,
        b1.reshape(1, MLP_W), W2, b2.reshape(1, MLP_W), W3.reshape(1, MLP_W),
        K, _E, sc)


# speedup vs baseline: 1.3843x; 1.0061x over previous
import functools
from itertools import combinations

import jax
import jax.numpy as jnp
import numpy as np
from jax import lax
from jax.experimental import pallas as pl
from jax.experimental.pallas import tpu as pltpu
from jax.experimental.pallas import tpu_sc as plsc

B, F, V, D = 4096, 26, 1000000, 16
FD = F * D
BF = B * F
MLP_W = 256
NUM_PAIRS = F * (F - 1) // 2

NC, NS = 2, 16
NW = NC * NS

NT_TILES = (V + 127) // 128
TPW = 244
NTS = 16
NBINS = 16
CAPB = 384
OUTW = NBINS * CAPB
CAP_TOT = NW * OUTW
M_CAP = 4608
SCCHUNK = 2048
NSCCH = BF // SCCHUNK
NCH_B = OUTW // 128
DUMPN = 104
DUMP0 = BF
CHUNK_W = BF // NW

_PAIR_ROWS, _PAIR_COLS = map(np.array, zip(*combinations(range(F), 2)))
_IOTA16 = np.arange(16, dtype=np.int32)


def _splat(x):
    return jnp.full((16,), x, dtype=jnp.int32)


def _sc_stream_body(idx_hbm, vt_hbm, mval_out, pos_out,
                    idx_st, idx_st2, r_list, p_list, rbin, pbin, slab,
                    vstage, sem_t, sem_t2, sem_i, sem_i2):
    wid = lax.axis_index("s") * NC + lax.axis_index("c")
    ja = wid * TPW
    jb = jnp.where(wid == NW - 1, NT_TILES, ja + TPW)
    iota = lax.iota(jnp.int32, 16)

    def scan_buf(buf, base, cnt):
        def scan_vec(i, cnt):
            rvec = buf[pl.ds(i * 16, 16)]
            j7 = lax.shift_right_logical(rvec, 7)
            m = (j7 >= _splat(ja)) & (j7 < _splat(jb))
            posvec = base + _splat(i * 16) + iota
            inc = plsc.cumsum(m.astype(jnp.int32))
            dest = jnp.minimum(cnt + inc - 1, _splat(M_CAP - 1))
            plsc.store_scatter(r_list, [dest], rvec, mask=m)
            plsc.store_scatter(p_list, [dest], posvec, mask=m)
            return cnt + plsc.all_reduce_population_count(m)

        return lax.fori_loop(0, SCCHUNK // 16, scan_vec, cnt, unroll=4)

    pltpu.sync_copy(idx_hbm.at[pl.ds(0, SCCHUNK)], idx_st)

    def scan_pair(c, cnt):
        @pl.when(c > 0)
        def _():
            pltpu.make_async_copy(idx_hbm.at[pl.ds(0, SCCHUNK)],
                                  idx_st, sem_i).wait()
        pltpu.async_copy(
            idx_hbm.at[pl.ds((2 * c + 1) * SCCHUNK, SCCHUNK)], idx_st2, sem_i2)
        cnt = scan_buf(idx_st, _splat(2 * c * SCCHUNK), cnt)

        @pl.when(c < NSCCH // 2 - 1)
        def _():
            pltpu.async_copy(
                idx_hbm.at[pl.ds((2 * c + 2) * SCCHUNK, SCCHUNK)], idx_st, sem_i)
        pltpu.make_async_copy(idx_hbm.at[pl.ds(0, SCCHUNK)],
                              idx_st2, sem_i2).wait()
        cnt = scan_buf(idx_st2, _splat((2 * c + 1) * SCCHUNK), cnt)
        return cnt

    cnt_vec = lax.fori_loop(0, NSCCH // 2, scan_pair, jnp.zeros((16,), jnp.int32))
    cnt = jnp.max(cnt_vec, axis=0)

    def fill_dump(i, carry):
        dv = _splat(DUMP0) + iota + _splat(16 * (i % 6))
        pbin[pl.ds(i * 16, 16)] = dv
        return carry

    lax.fori_loop(0, NBINS * CAPB // 16, fill_dump, 0)

    def bin_vec(i, cnts):
        rvec = r_list[pl.ds(i * 16, 16)]
        pvec = p_list[pl.ds(i * 16, 16)]
        valid = (_splat(i * 16) + iota) < cnt_vec
        bb = (lax.shift_right_logical(rvec, 7) - _splat(ja)) // NTS
        new = []
        for k in range(NBINS):
            ck = cnts[k]
            mk = valid & (bb == _splat(k))
            inc = plsc.cumsum(mk.astype(jnp.int32))
            dest = jnp.minimum(ck + inc - 1, _splat(CAPB - 1)) + _splat(k * CAPB)
            plsc.store_scatter(rbin, [dest], rvec, mask=mk)
            plsc.store_scatter(pbin, [dest], pvec, mask=mk)
            new.append(ck + plsc.all_reduce_population_count(mk))
        return tuple(new)

    cnts_vec = lax.fori_loop(0, (cnt + 15) // 16, bin_vec,
                             tuple(jnp.zeros((16,), jnp.int32)
                                   for _ in range(NBINS)))
    cnts = [jnp.max(cv, axis=0) for cv in cnts_vec]

    def fire(k, j0, half, sem):
        ntk = jnp.maximum(jnp.minimum(jb - j0, NTS), 0)

        def fire1(i, carry):
            jj = j0 + lax.shift_right_logical(i, 1)
            t = i & 1
            row = half * (NTS * 16) + (lax.shift_right_logical(i, 1) * 16 + t * 8)
            pltpu.async_copy(
                vt_hbm.at[pl.ds(t * 8, 8), pl.ds(pl.multiple_of(jj * 128, 128), 128)],
                slab.at[pl.ds(pl.multiple_of(row, 8), 8), :], sem)
            return carry

        lax.fori_loop(0, 2 * ntk, fire1, 0)
        return ntk

    def drain_n(ntk, sem):
        def drain1(i, carry):
            pltpu.make_async_copy(
                vt_hbm.at[pl.ds(0, 8), pl.ds(0, 128)],
                slab.at[pl.ds(0, 8), :], sem).wait()
            return carry

        lax.fori_loop(0, 2 * ntk, drain1, 0)

    ntks = [None] * NBINS
    ntks[0] = fire(0, ja, 0, sem_t)
    for k in range(NBINS):
        half = k % 2
        sem = sem_t if half == 0 else sem_t2
        if k + 1 < NBINS:
            nsem = sem_t2 if half == 0 else sem_t
            ntks[k + 1] = fire(k + 1, ja + (k + 1) * NTS, 1 - half, nsem)
        drain_n(ntks[k], sem)

        def extract(m, carry, k=k, half=half):
            rsp = plsc.load_gather(rbin, [_splat(k * CAPB + m)])
            jloc = lax.shift_right_logical(rsp, 7) - _splat(ja + k * NTS)
            rows = _splat(half * (NTS * 16)) + jloc * 16 + iota
            cols = rsp & _splat(127)
            v16 = plsc.load_gather(slab, [rows, cols])
            vstage[pl.ds(m * 16, 16)] = v16
            return carry

        lax.fori_loop(0, cnts[k], extract, 0)
        base = (wid * NBINS + k) * CAPB
        pltpu.sync_copy(vstage, mval_out.at[pl.ds(base * 16, CAPB * 16)])
        pltpu.sync_copy(pbin.at[pl.ds(k * CAPB, CAPB)],
                        pos_out.at[pl.ds(base, CAPB)])


@functools.cache
def _sc_stream():
    return pl.kernel(
        _sc_stream_body,
        out_type=[
            jax.ShapeDtypeStruct((CAP_TOT * 16,), jnp.float32),
            jax.ShapeDtypeStruct((CAP_TOT,), jnp.int32),
        ],
        mesh=plsc.VectorSubcoreMesh(core_axis_name="c", subcore_axis_name="s"),
        compiler_params=pltpu.CompilerParams(needs_layout_passes=False),
        scratch_types=[
            pltpu.VMEM((SCCHUNK,), jnp.int32),
            pltpu.VMEM((SCCHUNK,), jnp.int32),
            pltpu.VMEM((M_CAP,), jnp.int32),
            pltpu.VMEM((M_CAP,), jnp.int32),
            pltpu.VMEM((NBINS * CAPB,), jnp.int32),
            pltpu.VMEM((NBINS * CAPB,), jnp.int32),
            pltpu.VMEM((2 * NTS * 16, 128), jnp.float32),
            pltpu.VMEM((CAPB * 16,), jnp.float32),
            pltpu.SemaphoreType.DMA,
            pltpu.SemaphoreType.DMA,
            pltpu.SemaphoreType.DMA,
            pltpu.SemaphoreType.DMA,
        ],
    )


BLK = 12
NBLK = NCH_B // BLK


def _sc_scatter_body(mval_hbm, pos_hbm, idx_hbm, w16_hbm, val_hbm,
                     xv_out, lin_out,
                     pos_v, stv, idx_v, idxhi_v, w64_v, val_v, prod_v, lin_v,
                     sem_v, sem_g):
    wid = lax.axis_index("s") * NC + lax.axis_index("c")
    iota = lax.iota(jnp.int32, 16)
    fiota = iota.astype(jnp.float32)

    cbase = wid * CHUNK_W
    pltpu.sync_copy(idx_hbm.at[pl.ds(cbase, CHUNK_W)], idx_v)
    pltpu.sync_copy(val_hbm.at[pl.ds(cbase, CHUNK_W)], val_v)

    def prep(j, carry):
        off = pl.multiple_of(j * 16, 16)
        idxhi_v[pl.ds(off, 16)] = lax.shift_right_logical(idx_v[pl.ds(off, 16)], 4)
        return carry

    lax.fori_loop(0, CHUNK_W // 16, prep, 0, unroll=4)

    def gfire(j, carry):
        off = pl.multiple_of(j * 128, 128)
        pltpu.async_copy(w16_hbm.at[idxhi_v.at[pl.ds(off, 128)]],
                         w64_v.at[pl.ds(off, 128)], sem_g)
        return carry

    lax.fori_loop(0, CHUNK_W // 128, gfire, 0)

    pltpu.sync_copy(pos_hbm.at[pl.ds(wid * NCH_B, NCH_B)], pos_v)

    def block(b, carry):
        base = wid * OUTW + b * BLK * 128
        pltpu.sync_copy(mval_hbm.at[pl.ds(base, BLK * 128)], stv)

        def fire(i, c):
            off = pl.multiple_of(i * 128, 128)
            pltpu.async_copy(stv.at[pl.ds(off, 128)],
                             xv_out.at[pos_v.at[b * BLK + i]], sem_v)
            return c

        def drain(i, c):
            pltpu.make_async_copy(stv.at[pl.ds(0, 128)],
                                  xv_out.at[pl.ds(0, 128)], sem_v).wait()
            return c

        lax.fori_loop(0, BLK, fire, 0)
        lax.fori_loop(0, BLK, drain, 0)
        return carry

    lax.fori_loop(0, NBLK, block, 0)

    def gdrain(j, carry):
        pltpu.make_async_copy(w16_hbm.at[idxhi_v.at[pl.ds(0, 128)]],
                              w64_v.at[pl.ds(0, 128)], sem_g).wait()
        return carry

    lax.fori_loop(0, CHUNK_W // 128, gdrain, 0)

    def wmul(j, carry):
        off = pl.multiple_of(j * 16, 16)
        lanes = idx_v[pl.ds(off, 16)] & _splat(15)
        wv = plsc.load_gather(w64_v, [_splat(j * 16) + iota, lanes])
        prod_v[pl.ds(off, 16)] = wv * val_v[pl.ds(off, 16)]
        return carry

    lax.fori_loop(0, CHUNK_W // 16, wmul, 0, unroll=4)

    tailmask = (fiota < 10.0).astype(jnp.float32)
    lane0 = iota == 0

    def rowsum2(r, carry):
        s1 = prod_v[pl.ds(r * F, 16)]
        s2 = prod_v[pl.ds(r * F + 16, 16)] * tailmask
        tot = jnp.sum(s1 + s2, axis=0)
        plsc.store_scatter(lin_v, [_splat(r)],
                           jnp.full((16,), tot, jnp.float32), mask=lane0)
        return carry

    lax.fori_loop(0, CHUNK_W // F, rowsum2, 0)
    pltpu.sync_copy(lin_v, lin_out.at[pl.ds(wid * (CHUNK_W // F), CHUNK_W // F)])


@functools.cache
def _sc_scatter():
    return pl.kernel(
        _sc_scatter_body,
        out_type=[
            jax.ShapeDtypeStruct((BF + DUMPN, 16), jnp.float32),
            jax.ShapeDtypeStruct((B,), jnp.float32),
        ],
        mesh=plsc.VectorSubcoreMesh(core_axis_name="c", subcore_axis_name="s"),
        compiler_params=pltpu.CompilerParams(use_tc_tiling_on_sc=False,
                                             needs_layout_passes=False),
        scratch_types=[
            pltpu.VMEM((NCH_B, 128), jnp.int32),
            pltpu.VMEM((BLK * 128, 16), jnp.float32),
            pltpu.VMEM((CHUNK_W,), jnp.int32),
            pltpu.VMEM((CHUNK_W,), jnp.int32),
            pltpu.VMEM((CHUNK_W, 16), jnp.float32),
            pltpu.VMEM((CHUNK_W,), jnp.float32),
            pltpu.VMEM((CHUNK_W + 16,), jnp.float32),
            pltpu.VMEM((CHUNK_W // F,), jnp.float32),
            pltpu.SemaphoreType.DMA,
            pltpu.SemaphoreType.DMA,
        ],
    )


FDP = 512


def _sc_retile_body(in_hbm, out_hbm, buf1d, tiles, sem_o):
    wid = lax.axis_index("s") * NC + lax.axis_index("c")

    def group(g, carry):
        row0 = wid * 128 + g * 8

        @pl.when(g > 0)
        def _():
            for j in range(4):
                pltpu.make_async_copy(tiles.at[pl.ds(0, 8), :],
                                      out_hbm.at[pl.ds(0, 8), pl.ds(0, 128)],
                                      sem_o).wait()

        pltpu.sync_copy(in_hbm.at[pl.ds(pl.multiple_of(row0 * FD, 8), 8 * FD)],
                        buf1d)
        for j in range(4):
            for r in range(8):
                for c in range(8):
                    src = r * FD + j * 128 + c * 16
                    if src + 16 <= (r + 1) * FD:
                        tiles[j * 8 + r, pl.ds(c * 16, 16)] = \
                            buf1d[pl.ds(src, 16)]
                    else:
                        tiles[j * 8 + r, pl.ds(c * 16, 16)] = \
                            jnp.zeros((16,), jnp.float32)
        for j in range(4):
            pltpu.async_copy(
                tiles.at[pl.ds(j * 8, 8), :],
                out_hbm.at[pl.ds(pl.multiple_of(row0, 8), 8),
                           pl.ds(j * 128, 128)], sem_o)
        return carry

    lax.fori_loop(0, 16, group, 0)
    for j in range(4):
        pltpu.make_async_copy(tiles.at[pl.ds(0, 8), :],
                              out_hbm.at[pl.ds(0, 8), pl.ds(0, 128)],
                              sem_o).wait()


@functools.cache
def _sc_retile():
    return pl.kernel(
        _sc_retile_body,
        out_type=[jax.ShapeDtypeStruct((B, FDP), jnp.float32)],
        mesh=plsc.VectorSubcoreMesh(core_axis_name="c", subcore_axis_name="s"),
        compiler_params=pltpu.CompilerParams(needs_layout_passes=False),
        scratch_types=[
            pltpu.VMEM((8 * FD,), jnp.float32),
            pltpu.VMEM((32, 128), jnp.float32),
            pltpu.SemaphoreType.DMA,
        ],
    )


BB = 512
GRID = B // BB


def _tc_body(xv_ref, fv_ref, lin_ref, w0_ref, b0_ref, w1_ref, b1_ref,
             w2_ref, b2_ref, w3t_ref, k_ref, e_ref, sc_ref, out_ref):
    fv = fv_ref[...]
    val_exp = jnp.dot(fv, e_ref[...], preferred_element_type=jnp.float32)
    xv = xv_ref[...] * val_exp
    h = jnp.maximum(jnp.dot(xv, w0_ref[...], preferred_element_type=jnp.float32)
                    + b0_ref[...], 0.0)
    h = jnp.maximum(jnp.dot(h, w1_ref[...], preferred_element_type=jnp.float32)
                    + b1_ref[...], 0.0)
    h = jnp.maximum(jnp.dot(h, w2_ref[...], preferred_element_type=jnp.float32)
                    + b2_ref[...], 0.0)
    deep = jnp.sum(h * w3t_ref[...], axis=1, keepdims=True)
    z = jnp.dot(xv, k_ref[...], preferred_element_type=jnp.float32)
    fm = 0.5 * jnp.sum(xv * z, axis=1, keepdims=True)
    out_ref[...] = (lin_ref[...] + fm + deep) * sc_ref[0] + sc_ref[1]


_tc_call = pl.pallas_call(
    _tc_body,
    grid=(GRID,),
    in_specs=[
        pl.BlockSpec((BB, FDP), lambda i: (i, 0)),
        pl.BlockSpec((BB, F), lambda i: (i, 0)),
        pl.BlockSpec((BB, 1), lambda i: (i, 0)),
        pl.BlockSpec((FDP, MLP_W), lambda i: (0, 0)),
        pl.BlockSpec((1, MLP_W), lambda i: (0, 0)),
        pl.BlockSpec((MLP_W, MLP_W), lambda i: (0, 0)),
        pl.BlockSpec((1, MLP_W), lambda i: (0, 0)),
        pl.BlockSpec((MLP_W, MLP_W), lambda i: (0, 0)),
        pl.BlockSpec((1, MLP_W), lambda i: (0, 0)),
        pl.BlockSpec((1, MLP_W), lambda i: (0, 0)),
        pl.BlockSpec((FDP, FDP), lambda i: (0, 0)),
        pl.BlockSpec((F, FDP), lambda i: (0, 0)),
        pl.BlockSpec(memory_space=pltpu.SMEM),
    ],
    out_specs=pl.BlockSpec((BB, 1), lambda i: (i, 0)),
    out_shape=jax.ShapeDtypeStruct((B, 1), jnp.float32),
)

_E = np.zeros((F, FDP), dtype=np.float32)
_E[:, :FD] = np.kron(np.eye(F, dtype=np.float32), np.ones((1, D), dtype=np.float32))


def kernel(feat_index, feat_value, w_table, v_table, W0, b0, W1, b1, W2, b2,
           W3, b3, mask, bn_gamma, bn_beta, W_out, b_out):
    idx_flat = feat_index.reshape(BF)
    vt = v_table.T
    w16 = w_table.T.reshape(V // 16, 16)
    val_flat = feat_value.reshape(BF)
    mval, pos = _sc_stream()(idx_flat, vt)
    xv_all, lin_sc = _sc_scatter()(
        mval.reshape(CAP_TOT, 16), pos.reshape(NW * NCH_B, 128),
        idx_flat, w16, val_flat)
    (xv2,) = _sc_retile()(xv_all.reshape((BF + DUMPN) * 16))
    lin2 = lin_sc.reshape(B, 1)

    coef = (mask[0] * bn_gamma) * (1.0 / np.sqrt(1.0 + 1e-3))
    M = jnp.zeros((F, F), jnp.float32).at[_PAIR_ROWS, _PAIR_COLS].set(coef)
    M = M + M.T
    K = jnp.pad(jnp.kron(M, jnp.eye(D, dtype=jnp.float32)),
                ((0, FDP - FD), (0, FDP - FD)))
    scale = W_out[0, 0]
    shift = b_out[0] + scale * (b3[0] + jnp.sum(mask[0] * bn_beta))
    sc = jnp.stack([scale, shift])

    W0p = jnp.concatenate([W0, jnp.zeros((FDP - FD, MLP_W), jnp.float32)])
    return _tc_call(
        xv2, feat_value, lin2, W0p, b0.reshape(1, MLP_W), W1,
        b1.reshape(1, MLP_W), W2, b2.reshape(1, MLP_W), W3.reshape(1, MLP_W),
        K, _E, sc)
